# baseline (speedup 1.0000x reference)
import jax
import jax.numpy as jnp
from jax import lax
from jax.experimental import pallas as pl
from jax.experimental.pallas import tpu as pltpu

_N = 16777216
_C = 8192
_R = _N // _C
_BM = 256
_GRID = _R // _BM


def _fill_body(idx_ref, grad_ref, out_ref):
    pid = pl.program_id(0)
    out_ref[...] = jnp.zeros_like(out_ref)
    target = idx_ref[0]
    row = target // _C
    col = target % _C
    row0 = pid * _BM

    @pl.when((row >= row0) & (row < row0 + _BM))
    def _():
        r = row - row0
        cols = lax.broadcasted_iota(jnp.int32, (1, _C), 1)
        out_ref[pl.ds(r, 1), :] = jnp.where(cols == col, grad_ref[0], 0.0)


def kernel(grad_output, input_sizes, dim, index, out):
    n = out.shape[0]
    idx = ((jnp.asarray(index, jnp.int32) + jnp.asarray(dim, jnp.int32))
           % jnp.asarray(input_sizes, jnp.int32)).reshape((1,))
    gval = jnp.asarray(grad_output, jnp.float32).reshape((1,))
    res = pl.pallas_call(
        _fill_body,
        grid=(_GRID,),
        in_specs=[pl.BlockSpec(memory_space=pltpu.SMEM),
                  pl.BlockSpec(memory_space=pltpu.SMEM)],
        out_specs=pl.BlockSpec((_BM, _C), lambda i: (i, 0)),
        out_shape=jax.ShapeDtypeStruct((_R, _C), jnp.float32),
    )(idx, gval)
    return res.reshape(n)
